# baseline (speedup 1.0000x reference)
import functools

import jax
import jax.numpy as jnp
from jax import lax
from jax.experimental import pallas as pl
from jax.experimental.pallas import tpu as pltpu
from jax.experimental.pallas import tpu_sc as plsc

N = 10000
E = 320000
D = 128
H = 128
N_HOPS = 3

NC = 2
NS = 16
NW = NC * NS
EPT = E // NW
CH = 128
NFULL = EPT // CH
REM = EPT - NFULL * CH
NPAD = 10240
RPT = NPAD // NS
ZR = 64

_mesh = plsc.VectorSubcoreMesh(
    core_axis_name="c", subcore_axis_name="s", num_cores=NC, num_subcores=NS
)


def _fill(ref, rows, width, value):
    vec = jnp.full((16,), value, dtype=ref.dtype)

    def body(i, _):
        for l in range(width // 16):
            ref[i, pl.ds(l * 16, 16)] = vec
        return 0

    lax.fori_loop(0, rows, body, 0)


@functools.partial(
    pl.kernel,
    out_type=jax.ShapeDtypeStruct((NC, NPAD), jnp.float32),
    mesh=_mesh,
    scratch_types=[
        pltpu.VMEM((EPT,), jnp.int32),
        pltpu.VMEM((NPAD,), jnp.float32),
        pltpu.VMEM((NS, RPT), jnp.float32),
        pltpu.VMEM((RPT,), jnp.float32),
        pltpu.VMEM_SHARED((NS, NPAD), jnp.float32),
    ],
    compiler_params=pltpu.CompilerParams(needs_layout_passes=False),
)
def _deg_sc(dst_hbm, out_hbm, idx_v, cnt, red, osum, cnt_sh):
    c = lax.axis_index("c")
    s = lax.axis_index("s")
    w = c * NS + s

    zeros = jnp.zeros((16,), jnp.float32)

    def zero_body(i, _):
        cnt[pl.ds(i * 16, 16)] = zeros
        return 0

    lax.fori_loop(0, NPAD // 16, zero_body, 0)

    pltpu.sync_copy(dst_hbm.at[pl.ds(w * EPT, EPT)], idx_v)
    ones = jnp.ones((16,), jnp.float32)

    def add_body(j, _):
        iv = idx_v[pl.ds(j * 16, 16)]
        plsc.addupdate_scatter(cnt, [iv], ones)
        return 0

    lax.fori_loop(0, EPT // 16, add_body, 0)

    pltpu.sync_copy(cnt, cnt_sh.at[s])
    plsc.subcore_barrier()

    pltpu.sync_copy(cnt_sh.at[:, pl.ds(s * RPT, RPT)], red)

    def red_body(k, _):
        acc = red[0, pl.ds(k * 16, 16)]
        for t in range(1, NS):
            acc = acc + red[t, pl.ds(k * 16, 16)]
        osum[pl.ds(k * 16, 16)] = acc
        return 0

    lax.fori_loop(0, RPT // 16, red_body, 0)
    pltpu.sync_copy(osum, out_hbm.at[c].at[pl.ds(s * RPT, RPT)])


CH = 64
CPT = 156
EB = CPT * CH
TE0 = NW * EB
NTAIL = (E - TE0) // CH


@functools.partial(
    pl.kernel,
    out_type=jax.ShapeDtypeStruct((NC, NPAD, D), jnp.float32),
    mesh=_mesh,
    scratch_types=(
        [pltpu.VMEM((CH,), jnp.int32) for _ in range(2)]
        + [pltpu.VMEM((CH,), jnp.int32) for _ in range(4)]
        + [pltpu.VMEM((CH, D), jnp.float32) for _ in range(4)]
        + [pltpu.VMEM_SHARED((NPAD, D), jnp.float32)]
        + [pltpu.SemaphoreType.DMA for _ in range(14)]
    ),
)
def _hop_sc(g_hbm, src_hbm, dst_hbm, out_hbm,
            src0, src1, dst0, dst1, dst2, dst3,
            rows0, rows1, rows2, rows3, acc_sh,
            gsem0, gsem1, gsem2, gsem3, rsem0, rsem1,
            dsem0, dsem1, dsem2, dsem3, ssem0, ssem1, ssem2, ssem3):
    c = lax.axis_index("c")
    s = lax.axis_index("s")
    w = c * NS + s
    rows = (rows0, rows1, rows2, rows3)
    srci = (src0, src1)
    dsti = (dst0, dst1, dst2, dst3)
    gsem = (gsem0, gsem1, gsem2, gsem3)
    rsem = (rsem0, rsem1)
    dsem = (dsem0, dsem1, dsem2, dsem3)
    ssem = (ssem0, ssem1, ssem2, ssem3)

    def sidx_start(j, b):
        pltpu.async_copy(src_hbm.at[pl.ds(w * EB + j * CH, CH)], srci[b],
                         rsem[b])

    def sidx_wait(j, b):
        pltpu.make_async_copy(src_hbm.at[pl.ds(w * EB + j * CH, CH)],
                              srci[b], rsem[b]).wait()

    def didx_start(j, b):
        pltpu.async_copy(dst_hbm.at[pl.ds(w * EB + j * CH, CH)], dsti[b],
                         dsem[b])

    def didx_wait(j, b):
        pltpu.make_async_copy(dst_hbm.at[pl.ds(w * EB + j * CH, CH)],
                              dsti[b], dsem[b]).wait()

    def gather_start(sb, b):
        pltpu.async_copy(g_hbm.at[srci[sb]], rows[b], gsem[b])

    def gather_wait(sb, b):
        pltpu.make_async_copy(g_hbm.at[srci[sb]], rows[b], gsem[b]).wait()

    def scat_start(j, b):
        pltpu.async_copy(rows[b], acc_sh.at[dsti[b]], ssem[b], add=True)

    def scat_wait(j, b):
        pltpu.make_async_copy(rows[b], acc_sh.at[dsti[b]], ssem[b]).wait()

    sidx_start(0, 0)
    sidx_start(1, 1)
    didx_start(0, 0)
    didx_start(1, 1)
    _fill(rows3, CH, D, 0.0)
    for k in range(RPT // ZR):
        pltpu.async_copy(rows3, acc_sh.at[pl.ds(s * RPT + k * ZR, ZR)],
                         gsem3)
    sidx_wait(0, 0)
    gather_start(0, 0)
    for k in range(RPT // ZR):
        pltpu.make_async_copy(
            rows3, acc_sh.at[pl.ds(s * RPT + k * ZR, ZR)], gsem3).wait()
    plsc.subcore_barrier()

    def quad(k, _):
        for u in range(4):
            j = 4 * k + u
            gather_wait(u % 2, u)
            didx_wait(j, u)

            @pl.when(j >= 2)
            def _():
                scat_wait(j - 2, (u + 2) % 4)

            scat_start(j, u)

            @pl.when(j + 2 < CPT)
            def _():
                sidx_start(j + 2, u % 2)
                didx_start(j + 2, (u + 2) % 4)

            @pl.when(j + 1 < CPT)
            def _():
                sidx_wait(j + 1, (u + 1) % 2)
                gather_start((u + 1) % 2, (u + 1) % 4)
        return 0

    lax.fori_loop(0, CPT // 4, quad, 0)
    scat_wait(CPT - 2, (CPT - 2) % 4)
    scat_wait(CPT - 1, (CPT - 1) % 4)

    @pl.when(w < NTAIL)
    def _():
        base = TE0 + w * CH
        pltpu.sync_copy(src_hbm.at[pl.ds(base, CH)], src0)
        pltpu.sync_copy(dst_hbm.at[pl.ds(base, CH)], dst0)
        pltpu.async_copy(g_hbm.at[src0], rows0, gsem0).wait()
        pltpu.sync_copy(rows0, acc_sh.at[dst0], add=True)

    plsc.subcore_barrier()

    stb = (rows0, rows1)
    for k in range(RPT // ZR):
        r0 = s * RPT + k * ZR
        b = k % 2
        if k >= 2:
            pltpu.make_async_copy(
                stb[b], out_hbm.at[c].at[pl.ds(s * RPT + (k - 2) * ZR, ZR)],
                gsem[b]).wait()
        pltpu.sync_copy(acc_sh.at[pl.ds(r0, ZR)], stb[b])
        pltpu.async_copy(stb[b], out_hbm.at[c].at[pl.ds(r0, ZR)], gsem[b])
    for k in (RPT // ZR - 2, RPT // ZR - 1):
        b = k % 2
        pltpu.make_async_copy(
            stb[b], out_hbm.at[c].at[pl.ds(s * RPT + k * ZR, ZR)],
            gsem[b]).wait()


_RB = 1000


def _prep_body(degp_ref, x_ref, g0_ref, dinv2_ref, sq_ref):
    deg = 1.0 + degp_ref[0] + degp_ref[1]
    dinv = lax.rsqrt(deg)
    g0_ref[...] = x_ref[...] * dinv
    dinv2_ref[...] = dinv * dinv
    sq_ref[...] = deg * dinv


def _prep_tc(degp, x):
    return pl.pallas_call(
        _prep_body,
        grid=(N // _RB,),
        in_specs=[
            pl.BlockSpec((NC, _RB, 1), lambda i: (0, i, 0)),
            pl.BlockSpec((_RB, D), lambda i: (i, 0)),
        ],
        out_specs=[
            pl.BlockSpec((_RB, D), lambda i: (i, 0)),
            pl.BlockSpec((_RB, 1), lambda i: (i, 0)),
            pl.BlockSpec((_RB, 1), lambda i: (i, 0)),
        ],
        out_shape=[
            jax.ShapeDtypeStruct((N, D), jnp.float32),
            jax.ShapeDtypeStruct((N, 1), jnp.float32),
            jax.ShapeDtypeStruct((N, 1), jnp.float32),
        ],
    )(degp, x)


def _combine_body(p_ref, g_ref, dinv2_ref, out_ref):
    out_ref[...] = dinv2_ref[...] * (p_ref[0] + p_ref[1] + g_ref[...])


def _combine_tc(p, g, dinv2):
    return pl.pallas_call(
        _combine_body,
        grid=(N // _RB,),
        in_specs=[
            pl.BlockSpec((NC, _RB, D), lambda i: (0, i, 0)),
            pl.BlockSpec((_RB, D), lambda i: (i, 0)),
            pl.BlockSpec((_RB, 1), lambda i: (i, 0)),
        ],
        out_specs=pl.BlockSpec((_RB, D), lambda i: (i, 0)),
        out_shape=jax.ShapeDtypeStruct((N, D), jnp.float32),
    )(p, g, dinv2)


def _mlp_body(x_ref, g1_ref, g2_ref, p3_ref, dinv2_ref, sq_ref,
              W0_ref, W1_ref, W2_ref, W3_ref,
              b0_ref, b1_ref, b2_ref, b3_ref, Wf_ref, bf_ref, out_ref):
    sq = sq_ref[...]
    g2 = g2_ref[...]
    g3 = dinv2_ref[...] * (p3_ref[0] + p3_ref[1] + g2)
    feats = (x_ref[...], g1_ref[...] * sq, g2 * sq, g3 * sq)
    Ws = (W0_ref, W1_ref, W2_ref, W3_ref)
    bs = (b0_ref, b1_ref, b2_ref, b3_ref)
    acc = bf_ref[...]
    for i in range(N_HOPS + 1):
        ni = jax.nn.relu(
            jnp.dot(feats[i], Ws[i][...], preferred_element_type=jnp.float32)
            + bs[i][...]
        )
        acc = acc + jnp.dot(
            ni, Wf_ref[pl.ds(i * H, H), :], preferred_element_type=jnp.float32
        )
    out_ref[...] = jax.nn.relu(acc)


def _mlp_tc(x, g1, g2, p3, dinv2, sq, W0, W1, W2, W3, b0, b1, b2, b3, Wf, bf):
    row = lambda i: (i, 0)
    full2 = lambda i: (0, 0)
    return pl.pallas_call(
        _mlp_body,
        grid=(N // _RB,),
        in_specs=[
            pl.BlockSpec((_RB, D), row),
            pl.BlockSpec((_RB, D), row),
            pl.BlockSpec((_RB, D), row),
            pl.BlockSpec((NC, _RB, D), lambda i: (0, i, 0)),
            pl.BlockSpec((_RB, 1), row),
            pl.BlockSpec((_RB, 1), row),
            pl.BlockSpec((D, H), full2),
            pl.BlockSpec((D, H), full2),
            pl.BlockSpec((D, H), full2),
            pl.BlockSpec((D, H), full2),
            pl.BlockSpec((1, H), full2),
            pl.BlockSpec((1, H), full2),
            pl.BlockSpec((1, H), full2),
            pl.BlockSpec((1, H), full2),
            pl.BlockSpec((H * (N_HOPS + 1), H), full2),
            pl.BlockSpec((1, H), full2),
        ],
        out_specs=pl.BlockSpec((_RB, H), row),
        out_shape=jax.ShapeDtypeStruct((N, H), jnp.float32),
    )(x, g1, g2, p3, dinv2, sq, W0, W1, W2, W3, b0, b1, b2, b3, Wf, bf)


def kernel(x, edge_index, W0, W1, W2, W3, b0, b1, b2, b3, Wf, bf):
    src = edge_index[0].astype(jnp.int32)
    dst = edge_index[1].astype(jnp.int32)

    degp = _deg_sc(dst)
    g, dinv2, sq = _prep_tc(degp.reshape(NC, NPAD, 1), x)

    gs = []
    for _ in range(N_HOPS - 1):
        p = _hop_sc(g, src, dst)
        g = _combine_tc(p, g, dinv2)
        gs.append(g)
    p3 = _hop_sc(g, src, dst)

    return _mlp_tc(
        x, gs[0], gs[1], p3, dinv2, sq,
        W0, W1, W2, W3,
        b0.reshape(1, H), b1.reshape(1, H), b2.reshape(1, H), b3.reshape(1, H),
        Wf, bf.reshape(1, H),
    )


# speedup vs baseline: 1.2977x; 1.2977x over previous
import functools

import jax
import jax.numpy as jnp
from jax import lax
from jax.experimental import pallas as pl
from jax.experimental.pallas import tpu as pltpu
from jax.experimental.pallas import tpu_sc as plsc

N = 10000
E = 320000
D = 128
H = 128
N_HOPS = 3

NC = 2
NS = 16
NW = NC * NS
EPT = E // NW
CH = 128
NFULL = EPT // CH
REM = EPT - NFULL * CH
NPAD = 10240
RPT = NPAD // NS
ZR = 64

_mesh = plsc.VectorSubcoreMesh(
    core_axis_name="c", subcore_axis_name="s", num_cores=NC, num_subcores=NS
)


def _fill(ref, rows, width, value):
    vec = jnp.full((16,), value, dtype=ref.dtype)

    def body(i, _):
        for l in range(width // 16):
            ref[i, pl.ds(l * 16, 16)] = vec
        return 0

    lax.fori_loop(0, rows, body, 0)


@functools.partial(
    pl.kernel,
    out_type=jax.ShapeDtypeStruct((NC, NPAD), jnp.float32),
    mesh=_mesh,
    scratch_types=[
        pltpu.VMEM((EPT,), jnp.int32),
        pltpu.VMEM((NPAD,), jnp.float32),
        pltpu.VMEM((NS, RPT), jnp.float32),
        pltpu.VMEM((RPT,), jnp.float32),
        pltpu.VMEM_SHARED((NS, NPAD), jnp.float32),
    ],
    compiler_params=pltpu.CompilerParams(needs_layout_passes=False),
)
def _deg_sc(dst_hbm, out_hbm, idx_v, cnt, red, osum, cnt_sh):
    c = lax.axis_index("c")
    s = lax.axis_index("s")
    w = c * NS + s

    zeros = jnp.zeros((16,), jnp.float32)

    def zero_body(i, _):
        cnt[pl.ds(i * 16, 16)] = zeros
        return 0

    lax.fori_loop(0, NPAD // 16, zero_body, 0)

    pltpu.sync_copy(dst_hbm.at[pl.ds(w * EPT, EPT)], idx_v)
    ones = jnp.ones((16,), jnp.float32)

    def add_body(j, _):
        iv = idx_v[pl.ds(j * 16, 16)]
        plsc.addupdate_scatter(cnt, [iv], ones)
        return 0

    lax.fori_loop(0, EPT // 16, add_body, 0)

    pltpu.sync_copy(cnt, cnt_sh.at[s])
    plsc.subcore_barrier()

    pltpu.sync_copy(cnt_sh.at[:, pl.ds(s * RPT, RPT)], red)

    def red_body(k, _):
        acc = red[0, pl.ds(k * 16, 16)]
        for t in range(1, NS):
            acc = acc + red[t, pl.ds(k * 16, 16)]
        osum[pl.ds(k * 16, 16)] = acc
        return 0

    lax.fori_loop(0, RPT // 16, red_body, 0)
    pltpu.sync_copy(osum, out_hbm.at[c].at[pl.ds(s * RPT, RPT)])


CH = 128
CPT = (E // CH) // NW
EB = CPT * CH
TE0 = NW * EB
NTAIL = (E - TE0) // CH


@functools.partial(
    pl.kernel,
    out_type=jax.ShapeDtypeStruct((NC, NPAD, D), jnp.float32),
    mesh=_mesh,
    scratch_types=[
        pltpu.VMEM((CH,), jnp.int32),
        pltpu.VMEM((CH,), jnp.int32),
        pltpu.VMEM((CH,), jnp.int32),
        pltpu.VMEM((CH,), jnp.int32),
        pltpu.VMEM((CH,), jnp.int32),
        pltpu.VMEM((CH,), jnp.int32),
        pltpu.VMEM((CH, D), jnp.float32),
        pltpu.VMEM((CH, D), jnp.float32),
        pltpu.VMEM((ZR, D), jnp.float32),
        pltpu.VMEM_SHARED((NPAD, D), jnp.float32),
        pltpu.SemaphoreType.DMA,
        pltpu.SemaphoreType.DMA,
        pltpu.SemaphoreType.DMA,
        pltpu.SemaphoreType.DMA,
        pltpu.SemaphoreType.DMA,
        pltpu.SemaphoreType.DMA,
        pltpu.SemaphoreType.DMA,
        pltpu.SemaphoreType.DMA,
    ],
)
def _hop_sc(g_hbm, src_hbm, dst_hbm, out_hbm,
            src0, src1, dst0, dst1, srct, dstt, rows0, rows1, stage0,
            acc_sh, gsem0, gsem1, ssem0, ssem1, dsem0, dsem1, rsem0, rsem1):
    c = lax.axis_index("c")
    s = lax.axis_index("s")
    w = c * NS + s
    rows = (rows0, rows1)
    srci = (src0, src1)
    dsti = (dst0, dst1)
    gsem = (gsem0, gsem1)
    ssem = (ssem0, ssem1)
    dsem = (dsem0, dsem1)
    rsem = (rsem0, rsem1)

    def sidx_start(j, b):
        pltpu.async_copy(src_hbm.at[pl.ds(w * EB + j * CH, CH)], srci[b],
                         rsem[b])

    def sidx_wait(j, b):
        pltpu.make_async_copy(src_hbm.at[pl.ds(w * EB + j * CH, CH)],
                              srci[b], rsem[b]).wait()

    def didx_start(j, b):
        pltpu.async_copy(dst_hbm.at[pl.ds(w * EB + j * CH, CH)], dsti[b],
                         dsem[b])

    def didx_wait(j, b):
        pltpu.make_async_copy(dst_hbm.at[pl.ds(w * EB + j * CH, CH)],
                              dsti[b], dsem[b]).wait()

    def gather_start(j, b):
        pltpu.async_copy(g_hbm.at[srci[b]], rows[b], gsem[b])

    def gather_wait(j, b):
        pltpu.make_async_copy(g_hbm.at[srci[b]], rows[b], gsem[b]).wait()

    def scat_start(j, b):
        pltpu.async_copy(rows[b], acc_sh.at[dsti[b]], ssem[b], add=True)

    def scat_wait(j, b):
        pltpu.make_async_copy(rows[b], acc_sh.at[dsti[b]], ssem[b]).wait()

    sidx_start(0, 0)
    didx_start(0, 0)
    sidx_start(1, 1)
    _fill(stage0, ZR, D, 0.0)
    for k in range(RPT // ZR):
        pltpu.async_copy(stage0, acc_sh.at[pl.ds(s * RPT + k * ZR, ZR)],
                         ssem0)
    sidx_wait(0, 0)
    gather_start(0, 0)
    for k in range(RPT // ZR):
        pltpu.make_async_copy(
            stage0, acc_sh.at[pl.ds(s * RPT + k * ZR, ZR)], ssem0).wait()
    plsc.subcore_barrier()

    def pair(k, _):
        for b in range(2):
            j = 2 * k + b
            gather_wait(j, b)
            didx_wait(j, b)
            scat_start(j, b)

            @pl.when(j >= 1)
            def _():
                scat_wait(j - 1, 1 - b)

            @pl.when(j + 1 < CPT)
            def _():
                didx_start(j + 1, 1 - b)
                sidx_wait(j + 1, 1 - b)
                gather_start(j + 1, 1 - b)

            @pl.when(j + 2 < CPT)
            def _():
                sidx_start(j + 2, b)
        return 0

    lax.fori_loop(0, CPT // 2, pair, 0)
    scat_wait(CPT - 1, (CPT - 1) % 2)

    @pl.when(w < NTAIL)
    def _():
        base = TE0 + w * CH
        pltpu.sync_copy(src_hbm.at[pl.ds(base, CH)], srct)
        pltpu.sync_copy(dst_hbm.at[pl.ds(base, CH)], dstt)
        pltpu.async_copy(g_hbm.at[srct], rows0, gsem0).wait()
        pltpu.sync_copy(rows0, acc_sh.at[dstt], add=True)

    plsc.subcore_barrier()

    stb = (stage0, rows0.at[pl.ds(0, ZR)])
    for k in range(RPT // ZR):
        r0 = s * RPT + k * ZR
        b = k % 2
        if k >= 2:
            pltpu.make_async_copy(
                stb[b], out_hbm.at[c].at[pl.ds(s * RPT + (k - 2) * ZR, ZR)],
                gsem[b]).wait()
        pltpu.sync_copy(acc_sh.at[pl.ds(r0, ZR)], stb[b])
        pltpu.async_copy(stb[b], out_hbm.at[c].at[pl.ds(r0, ZR)], gsem[b])
    for k in (RPT // ZR - 2, RPT // ZR - 1):
        b = k % 2
        pltpu.make_async_copy(
            stb[b], out_hbm.at[c].at[pl.ds(s * RPT + k * ZR, ZR)],
            gsem[b]).wait()


_RB = 1000


def _prep_body(degp_ref, x_ref, g0_ref, dinv2_ref, sq_ref):
    deg = 1.0 + degp_ref[0] + degp_ref[1]
    dinv = lax.rsqrt(deg)
    g0_ref[...] = x_ref[...] * dinv
    dinv2_ref[...] = dinv * dinv
    sq_ref[...] = deg * dinv


def _prep_tc(degp, x):
    return pl.pallas_call(
        _prep_body,
        grid=(N // _RB,),
        in_specs=[
            pl.BlockSpec((NC, _RB, 1), lambda i: (0, i, 0)),
            pl.BlockSpec((_RB, D), lambda i: (i, 0)),
        ],
        out_specs=[
            pl.BlockSpec((_RB, D), lambda i: (i, 0)),
            pl.BlockSpec((_RB, 1), lambda i: (i, 0)),
            pl.BlockSpec((_RB, 1), lambda i: (i, 0)),
        ],
        out_shape=[
            jax.ShapeDtypeStruct((N, D), jnp.float32),
            jax.ShapeDtypeStruct((N, 1), jnp.float32),
            jax.ShapeDtypeStruct((N, 1), jnp.float32),
        ],
    )(degp, x)


def _combine_body(p_ref, g_ref, dinv2_ref, out_ref):
    out_ref[...] = dinv2_ref[...] * (p_ref[0] + p_ref[1] + g_ref[...])


def _combine_tc(p, g, dinv2):
    return pl.pallas_call(
        _combine_body,
        grid=(N // _RB,),
        in_specs=[
            pl.BlockSpec((NC, _RB, D), lambda i: (0, i, 0)),
            pl.BlockSpec((_RB, D), lambda i: (i, 0)),
            pl.BlockSpec((_RB, 1), lambda i: (i, 0)),
        ],
        out_specs=pl.BlockSpec((_RB, D), lambda i: (i, 0)),
        out_shape=jax.ShapeDtypeStruct((N, D), jnp.float32),
    )(p, g, dinv2)


def _mlp_body(x_ref, g1_ref, g2_ref, p3_ref, dinv2_ref, sq_ref,
              W0_ref, W1_ref, W2_ref, W3_ref,
              b0_ref, b1_ref, b2_ref, b3_ref, Wf_ref, bf_ref, out_ref):
    sq = sq_ref[...]
    g2 = g2_ref[...]
    g3 = dinv2_ref[...] * (p3_ref[0] + p3_ref[1] + g2)
    feats = (x_ref[...], g1_ref[...] * sq, g2 * sq, g3 * sq)
    Ws = (W0_ref, W1_ref, W2_ref, W3_ref)
    bs = (b0_ref, b1_ref, b2_ref, b3_ref)
    acc = bf_ref[...]
    for i in range(N_HOPS + 1):
        ni = jax.nn.relu(
            jnp.dot(feats[i], Ws[i][...], preferred_element_type=jnp.float32)
            + bs[i][...]
        )
        acc = acc + jnp.dot(
            ni, Wf_ref[pl.ds(i * H, H), :], preferred_element_type=jnp.float32
        )
    out_ref[...] = jax.nn.relu(acc)


def _mlp_tc(x, g1, g2, p3, dinv2, sq, W0, W1, W2, W3, b0, b1, b2, b3, Wf, bf):
    row = lambda i: (i, 0)
    full2 = lambda i: (0, 0)
    return pl.pallas_call(
        _mlp_body,
        grid=(N // _RB,),
        in_specs=[
            pl.BlockSpec((_RB, D), row),
            pl.BlockSpec((_RB, D), row),
            pl.BlockSpec((_RB, D), row),
            pl.BlockSpec((NC, _RB, D), lambda i: (0, i, 0)),
            pl.BlockSpec((_RB, 1), row),
            pl.BlockSpec((_RB, 1), row),
            pl.BlockSpec((D, H), full2),
            pl.BlockSpec((D, H), full2),
            pl.BlockSpec((D, H), full2),
            pl.BlockSpec((D, H), full2),
            pl.BlockSpec((1, H), full2),
            pl.BlockSpec((1, H), full2),
            pl.BlockSpec((1, H), full2),
            pl.BlockSpec((1, H), full2),
            pl.BlockSpec((H * (N_HOPS + 1), H), full2),
            pl.BlockSpec((1, H), full2),
        ],
        out_specs=pl.BlockSpec((_RB, H), row),
        out_shape=jax.ShapeDtypeStruct((N, H), jnp.float32),
    )(x, g1, g2, p3, dinv2, sq, W0, W1, W2, W3, b0, b1, b2, b3, Wf, bf)


def kernel(x, edge_index, W0, W1, W2, W3, b0, b1, b2, b3, Wf, bf):
    src = edge_index[0].astype(jnp.int32)
    dst = edge_index[1].astype(jnp.int32)

    degp = _deg_sc(dst)
    g, dinv2, sq = _prep_tc(degp.reshape(NC, NPAD, 1), x)

    gs = []
    for _ in range(N_HOPS - 1):
        p = _hop_sc(g, src, dst)
        g = _combine_tc(p, g, dinv2)
        gs.append(g)
    p3 = _hop_sc(g, src, dst)

    return _mlp_tc(
        x, gs[0], gs[1], p3, dinv2, sq,
        W0, W1, W2, W3,
        b0.reshape(1, H), b1.reshape(1, H), b2.reshape(1, H), b3.reshape(1, H),
        Wf, bf.reshape(1, H),
    )


# speedup vs baseline: 1.2986x; 1.0007x over previous
import functools

import jax
import jax.numpy as jnp
from jax import lax
from jax.experimental import pallas as pl
from jax.experimental.pallas import tpu as pltpu
from jax.experimental.pallas import tpu_sc as plsc

N = 10000
E = 320000
D = 128
H = 128
N_HOPS = 3

NC = 2
NS = 16
NW = NC * NS
EPT = E // NW
NPAD = 10240
RPT = NPAD // NS
ZR = 64

_mesh = plsc.VectorSubcoreMesh(
    core_axis_name="c", subcore_axis_name="s", num_cores=NC, num_subcores=NS
)


def _fill(ref, rows, width, value):
    vec = jnp.full((16,), value, dtype=ref.dtype)

    def body(i, _):
        for l in range(width // 16):
            ref[i, pl.ds(l * 16, 16)] = vec
        return 0

    lax.fori_loop(0, rows, body, 0)


@functools.partial(
    pl.kernel,
    out_type=jax.ShapeDtypeStruct((NC, NPAD), jnp.float32),
    mesh=_mesh,
    scratch_types=[
        pltpu.VMEM((EPT,), jnp.int32),
        pltpu.VMEM((NPAD,), jnp.float32),
        pltpu.VMEM((NS, RPT), jnp.float32),
        pltpu.VMEM((RPT,), jnp.float32),
        pltpu.VMEM_SHARED((NS, NPAD), jnp.float32),
    ],
    compiler_params=pltpu.CompilerParams(needs_layout_passes=False),
)
def _deg_sc(dst_hbm, out_hbm, idx_v, cnt, red, osum, cnt_sh):
    c = lax.axis_index("c")
    s = lax.axis_index("s")
    w = c * NS + s

    zeros = jnp.zeros((16,), jnp.float32)

    def zero_body(i, _):
        cnt[pl.ds(i * 16, 16)] = zeros
        return 0

    lax.fori_loop(0, NPAD // 16, zero_body, 0)

    pltpu.sync_copy(dst_hbm.at[pl.ds(w * EPT, EPT)], idx_v)
    ones = jnp.ones((16,), jnp.float32)

    def add_body(j, _):
        iv = idx_v[pl.ds(j * 16, 16)]
        plsc.addupdate_scatter(cnt, [iv], ones)
        return 0

    lax.fori_loop(0, EPT // 16, add_body, 0)

    pltpu.sync_copy(cnt, cnt_sh.at[s])
    plsc.subcore_barrier()

    pltpu.sync_copy(cnt_sh.at[:, pl.ds(s * RPT, RPT)], red)

    def red_body(k, _):
        acc = red[0, pl.ds(k * 16, 16)]
        for t in range(1, NS):
            acc = acc + red[t, pl.ds(k * 16, 16)]
        osum[pl.ds(k * 16, 16)] = acc
        return 0

    lax.fori_loop(0, RPT // 16, red_body, 0)
    pltpu.sync_copy(osum, out_hbm.at[c].at[pl.ds(s * RPT, RPT)])


CH = 128
CPT = (E // CH) // NW
EB = CPT * CH
TE0 = NW * EB
NTAIL = (E - TE0) // CH


@functools.partial(
    pl.kernel,
    out_type=jax.ShapeDtypeStruct((NC, NPAD, D), jnp.float32),
    mesh=_mesh,
    scratch_types=[
        pltpu.VMEM((CH,), jnp.int32),
        pltpu.VMEM((CH,), jnp.int32),
        pltpu.VMEM((CH,), jnp.int32),
        pltpu.VMEM((CH,), jnp.int32),
        pltpu.VMEM((CH,), jnp.int32),
        pltpu.VMEM((CH,), jnp.int32),
        pltpu.VMEM((CH, D), jnp.float32),
        pltpu.VMEM((CH, D), jnp.float32),
        pltpu.VMEM((ZR, D), jnp.float32),
        pltpu.VMEM_SHARED((NPAD, D), jnp.float32),
        pltpu.SemaphoreType.DMA,
        pltpu.SemaphoreType.DMA,
        pltpu.SemaphoreType.DMA,
        pltpu.SemaphoreType.DMA,
        pltpu.SemaphoreType.DMA,
        pltpu.SemaphoreType.DMA,
        pltpu.SemaphoreType.DMA,
        pltpu.SemaphoreType.DMA,
    ],
)
def _hop_sc(g_hbm, src_hbm, dst_hbm, out_hbm,
            src0, src1, dst0, dst1, srct, dstt, rows0, rows1, stage0,
            acc_sh, gsem0, gsem1, ssem0, ssem1, dsem0, dsem1, rsem0, rsem1):
    c = lax.axis_index("c")
    s = lax.axis_index("s")
    w = c * NS + s
    rows = (rows0, rows1)
    srci = (src0, src1)
    dsti = (dst0, dst1)
    gsem = (gsem0, gsem1)
    ssem = (ssem0, ssem1)
    dsem = (dsem0, dsem1)
    rsem = (rsem0, rsem1)

    def sidx_start(j, b):
        pltpu.async_copy(src_hbm.at[pl.ds(w * EB + j * CH, CH)], srci[b],
                         rsem[b])

    def sidx_wait(j, b):
        pltpu.make_async_copy(src_hbm.at[pl.ds(w * EB + j * CH, CH)],
                              srci[b], rsem[b]).wait()

    def didx_start(j, b):
        pltpu.async_copy(dst_hbm.at[pl.ds(w * EB + j * CH, CH)], dsti[b],
                         dsem[b])

    def didx_wait(j, b):
        pltpu.make_async_copy(dst_hbm.at[pl.ds(w * EB + j * CH, CH)],
                              dsti[b], dsem[b]).wait()

    def gather_start(j, b):
        pltpu.async_copy(g_hbm.at[srci[b]], rows[b], gsem[b])

    def gather_wait(j, b):
        pltpu.make_async_copy(g_hbm.at[srci[b]], rows[b], gsem[b]).wait()

    def scat_start(j, b):
        pltpu.async_copy(rows[b], acc_sh.at[dsti[b]], ssem[b], add=True)

    def scat_wait(j, b):
        pltpu.make_async_copy(rows[b], acc_sh.at[dsti[b]], ssem[b]).wait()

    sidx_start(0, 0)
    didx_start(0, 0)
    sidx_start(1, 1)
    _fill(stage0, ZR, D, 0.0)
    for k in range(RPT // ZR):
        pltpu.async_copy(stage0, acc_sh.at[pl.ds(s * RPT + k * ZR, ZR)],
                         ssem0)
    sidx_wait(0, 0)
    gather_start(0, 0)
    for k in range(RPT // ZR):
        pltpu.make_async_copy(
            stage0, acc_sh.at[pl.ds(s * RPT + k * ZR, ZR)], ssem0).wait()
    plsc.subcore_barrier()

    def pair(k, _):
        for b in range(2):
            j = 2 * k + b
            gather_wait(j, b)
            didx_wait(j, b)
            scat_start(j, b)

            @pl.when(j >= 1)
            def _():
                scat_wait(j - 1, 1 - b)

            @pl.when(j + 1 < CPT)
            def _():
                didx_start(j + 1, 1 - b)
                sidx_wait(j + 1, 1 - b)
                gather_start(j + 1, 1 - b)

            @pl.when(j + 2 < CPT)
            def _():
                sidx_start(j + 2, b)
        return 0

    lax.fori_loop(0, CPT // 2, pair, 0)
    scat_wait(CPT - 1, (CPT - 1) % 2)

    @pl.when(w < NTAIL)
    def _():
        base = TE0 + w * CH
        pltpu.sync_copy(src_hbm.at[pl.ds(base, CH)], srct)
        pltpu.sync_copy(dst_hbm.at[pl.ds(base, CH)], dstt)
        pltpu.async_copy(g_hbm.at[srct], rows0, gsem0).wait()
        pltpu.sync_copy(rows0, acc_sh.at[dstt], add=True)

    plsc.subcore_barrier()

    stb = (stage0, rows0.at[pl.ds(0, ZR)])
    for k in range(RPT // ZR):
        r0 = s * RPT + k * ZR
        b = k % 2
        if k >= 2:
            pltpu.make_async_copy(
                stb[b], out_hbm.at[c].at[pl.ds(s * RPT + (k - 2) * ZR, ZR)],
                gsem[b]).wait()
        pltpu.sync_copy(acc_sh.at[pl.ds(r0, ZR)], stb[b])
        pltpu.async_copy(stb[b], out_hbm.at[c].at[pl.ds(r0, ZR)], gsem[b])
    for k in (RPT // ZR - 2, RPT // ZR - 1):
        b = k % 2
        pltpu.make_async_copy(
            stb[b], out_hbm.at[c].at[pl.ds(s * RPT + k * ZR, ZR)],
            gsem[b]).wait()


_RB = 1000


def _prep_body(degp_ref, x_ref, g0_ref, dinv2_ref, sq_ref):
    deg = 1.0 + degp_ref[0] + degp_ref[1]
    dinv = lax.rsqrt(deg)
    g0_ref[...] = x_ref[...] * dinv
    dinv2_ref[...] = dinv * dinv
    sq_ref[...] = deg * dinv


def _prep_tc(degp, x):
    return pl.pallas_call(
        _prep_body,
        grid=(N // _RB,),
        in_specs=[
            pl.BlockSpec((NC, _RB, 1), lambda i: (0, i, 0)),
            pl.BlockSpec((_RB, D), lambda i: (i, 0)),
        ],
        out_specs=[
            pl.BlockSpec((_RB, D), lambda i: (i, 0)),
            pl.BlockSpec((_RB, 1), lambda i: (i, 0)),
            pl.BlockSpec((_RB, 1), lambda i: (i, 0)),
        ],
        out_shape=[
            jax.ShapeDtypeStruct((N, D), jnp.float32),
            jax.ShapeDtypeStruct((N, 1), jnp.float32),
            jax.ShapeDtypeStruct((N, 1), jnp.float32),
        ],
    )(degp, x)


def _combine_body(p_ref, g_ref, dinv2_ref, out_ref):
    out_ref[...] = dinv2_ref[...] * (p_ref[0] + p_ref[1] + g_ref[...])


def _combine_tc(p, g, dinv2):
    return pl.pallas_call(
        _combine_body,
        grid=(N // _RB,),
        in_specs=[
            pl.BlockSpec((NC, _RB, D), lambda i: (0, i, 0)),
            pl.BlockSpec((_RB, D), lambda i: (i, 0)),
            pl.BlockSpec((_RB, 1), lambda i: (i, 0)),
        ],
        out_specs=pl.BlockSpec((_RB, D), lambda i: (i, 0)),
        out_shape=jax.ShapeDtypeStruct((N, D), jnp.float32),
    )(p, g, dinv2)


def _mlp_body(x_ref, g1_ref, g2_ref, p3_ref, dinv2_ref, sq_ref,
              W0_ref, W1_ref, W2_ref, W3_ref,
              b0_ref, b1_ref, b2_ref, b3_ref, Wf_ref, bf_ref, out_ref):
    sq = sq_ref[...]
    g2 = g2_ref[...]
    g3 = dinv2_ref[...] * (p3_ref[0] + p3_ref[1] + g2)
    feats = (x_ref[...], g1_ref[...] * sq, g2 * sq, g3 * sq)
    Ws = (W0_ref, W1_ref, W2_ref, W3_ref)
    bs = (b0_ref, b1_ref, b2_ref, b3_ref)
    acc = bf_ref[...]
    for i in range(N_HOPS + 1):
        ni = jax.nn.relu(
            jnp.dot(feats[i], Ws[i][...], preferred_element_type=jnp.float32)
            + bs[i][...]
        )
        acc = acc + jnp.dot(
            ni, Wf_ref[pl.ds(i * H, H), :], preferred_element_type=jnp.float32
        )
    out_ref[...] = jax.nn.relu(acc)


def _mlp_tc(x, g1, g2, p3, dinv2, sq, W0, W1, W2, W3, b0, b1, b2, b3, Wf, bf):
    row = lambda i: (i, 0)
    full2 = lambda i: (0, 0)
    return pl.pallas_call(
        _mlp_body,
        grid=(N // _RB,),
        in_specs=[
            pl.BlockSpec((_RB, D), row),
            pl.BlockSpec((_RB, D), row),
            pl.BlockSpec((_RB, D), row),
            pl.BlockSpec((NC, _RB, D), lambda i: (0, i, 0)),
            pl.BlockSpec((_RB, 1), row),
            pl.BlockSpec((_RB, 1), row),
            pl.BlockSpec((D, H), full2),
            pl.BlockSpec((D, H), full2),
            pl.BlockSpec((D, H), full2),
            pl.BlockSpec((D, H), full2),
            pl.BlockSpec((1, H), full2),
            pl.BlockSpec((1, H), full2),
            pl.BlockSpec((1, H), full2),
            pl.BlockSpec((1, H), full2),
            pl.BlockSpec((H * (N_HOPS + 1), H), full2),
            pl.BlockSpec((1, H), full2),
        ],
        out_specs=pl.BlockSpec((_RB, H), row),
        out_shape=jax.ShapeDtypeStruct((N, H), jnp.float32),
    )(x, g1, g2, p3, dinv2, sq, W0, W1, W2, W3, b0, b1, b2, b3, Wf, bf)


def kernel(x, edge_index, W0, W1, W2, W3, b0, b1, b2, b3, Wf, bf):
    src = edge_index[0].astype(jnp.int32)
    dst = edge_index[1].astype(jnp.int32)

    degp = _deg_sc(dst)
    g, dinv2, sq = _prep_tc(degp.reshape(NC, NPAD, 1), x)

    gs = []
    for _ in range(N_HOPS - 1):
        p = _hop_sc(g, src, dst)
        g = _combine_tc(p, g, dinv2)
        gs.append(g)
    p3 = _hop_sc(g, src, dst)

    return _mlp_tc(
        x, gs[0], gs[1], p3, dinv2, sq,
        W0, W1, W2, W3,
        b0.reshape(1, H), b1.reshape(1, H), b2.reshape(1, H), b3.reshape(1, H),
        Wf, bf.reshape(1, H),
    )
